# baseline (speedup 1.0000x reference)
import functools

import jax
import jax.numpy as jnp
from jax import lax
from jax.experimental import pallas as pl
from jax.experimental.pallas import tpu as pltpu
from jax.experimental.pallas import tpu_sc as plsc

NC = 2
NS = 16
CH = 128
BLK = 3


def _round_up(x, m):
    return (x + m - 1) // m * m


def _mesh():
    return plsc.VectorSubcoreMesh(
        core_axis_name="c", subcore_axis_name="s",
        num_cores=NC, num_subcores=NS)


def _sc_embed(z_pad, emb0, emb1):
    n_pad, h = z_pad.shape[0], emb0.shape[1]
    chunks = n_pad // (NS * CH)
    cps = chunks * CH

    @functools.partial(
        pl.kernel,
        out_type=[jax.ShapeDtypeStruct((n_pad, h), jnp.float32),
                  jax.ShapeDtypeStruct((n_pad, h), jnp.float32)],
        mesh=_mesh(),
        compiler_params=pltpu.CompilerParams(use_tc_tiling_on_sc=False),
        scratch_types=[pltpu.VMEM((cps,), jnp.int32),
                       pltpu.VMEM((CH, h), jnp.float32),
                       pltpu.VMEM((CH, h), jnp.float32),
                       pltpu.SemaphoreType.DMA,
                       pltpu.SemaphoreType.DMA,
                       pltpu.SemaphoreType.DMA,
                       pltpu.SemaphoreType.DMA],
    )
    def k(z_ref, e0_ref, e1_ref, x0_ref, x1_ref, idx_v, r0, r1,
          sg0, sg1, so0, so1):
        c = lax.axis_index("c")
        s = lax.axis_index("s")
        base = s * cps

        def body(e_ref, x_ref):
            pltpu.sync_copy(z_ref.at[pl.ds(base, cps)], idx_v)

            @pl.loop(0, chunks, step=2)
            def _(j):
                g0 = pltpu.async_copy(
                    e_ref.at[idx_v.at[pl.ds(j * CH, CH)]], r0, sg0)
                g1 = pltpu.async_copy(
                    e_ref.at[idx_v.at[pl.ds((j + 1) * CH, CH)]], r1, sg1)
                g0.wait()
                w0 = pltpu.async_copy(
                    r0, x_ref.at[pl.ds(base + j * CH, CH)], so0)
                g1.wait()
                w1 = pltpu.async_copy(
                    r1, x_ref.at[pl.ds(base + (j + 1) * CH, CH)], so1)
                w0.wait()
                w1.wait()

        @pl.when(c == 0)
        def _():
            body(e0_ref, x0_ref)

        @pl.when(c == 1)
        def _():
            body(e1_ref, x1_ref)

    return k(z_pad, emb0, emb1)


def _sc_agg(x0, x1, row2d, col2d, n, sp_rows):
    h = x0.shape[1]
    rps = row2d.shape[0] // NS
    nbl = rps // BLK - 1
    nout = n // NS

    @functools.partial(
        pl.kernel,
        out_type=[jax.ShapeDtypeStruct((n, h), jnp.float32),
                  jax.ShapeDtypeStruct((n, h), jnp.float32)],
        mesh=_mesh(),
        compiler_params=pltpu.CompilerParams(use_tc_tiling_on_sc=False),
        scratch_types=[pltpu.VMEM((BLK, CH), jnp.int32),
                       pltpu.VMEM((BLK, CH), jnp.int32),
                       pltpu.VMEM((BLK, CH), jnp.int32),
                       pltpu.VMEM((BLK, CH), jnp.int32),
                       pltpu.VMEM((BLK, CH, h), jnp.float32),
                       pltpu.VMEM((BLK, CH, h), jnp.float32),
                       pltpu.VMEM_SHARED((sp_rows, h), jnp.float32),
                       pltpu.SemaphoreType.DMA,
                       pltpu.SemaphoreType.DMA,
                       pltpu.SemaphoreType.DMA,
                       pltpu.SemaphoreType.DMA],
    )
    def k(x0_ref, x1_ref, row_ref, col_ref, o0_ref, o1_ref,
          r0b, c0b, r1b, c1b, gb0, gb1, acc,
          sem_g0, sem_g1, sem_s0, sem_s1):
        c = lax.axis_index("c")
        s = lax.axis_index("s")

        def body(x_ref, o_ref):
            pltpu.sync_copy(x_ref.at[pl.ds(s * nout, nout)],
                            acc.at[pl.ds(s * nout, nout)])
            plsc.subcore_barrier()

            base = s * rps

            def load_idx(b, rb, cb):
                pltpu.sync_copy(row_ref.at[pl.ds(base + b * BLK, BLK)], rb)
                pltpu.sync_copy(col_ref.at[pl.ds(base + b * BLK, BLK)], cb)

            def fire_g(cb, gb, sem):
                return [pltpu.async_copy(x_ref.at[cb.at[j]], gb.at[j], sem)
                        for j in range(BLK)]

            def fire_s(rb, gb, sem):
                return [pltpu.async_copy(gb.at[j], acc.at[rb.at[j]], sem,
                                         add=True)
                        for j in range(BLK)]

            load_idx(0, r0b, c0b)

            @pl.loop(0, nbl, step=2)
            def _(i):
                g0 = fire_g(c0b, gb0, sem_g0)
                load_idx(i + 1, r1b, c1b)
                g1 = fire_g(c1b, gb1, sem_g1)
                for dsc in g0:
                    dsc.wait()
                s0 = fire_s(r0b, gb0, sem_s0)
                for dsc in s0:
                    dsc.wait()
                load_idx(i + 2, r0b, c0b)
                for dsc in g1:
                    dsc.wait()
                s1 = fire_s(r1b, gb1, sem_s1)
                for dsc in s1:
                    dsc.wait()

            plsc.subcore_barrier()
            pltpu.sync_copy(acc.at[pl.ds(s * nout, nout)],
                            o_ref.at[pl.ds(s * nout, nout)])

        @pl.when(c == 0)
        def _():
            body(x0_ref, o0_ref)

        @pl.when(c == 1)
        def _():
            body(x1_ref, o1_ref)

    return k(x0, x1, row2d, col2d)


def _tc_linear(s0, s1, w, b, final):
    n, h = s0.shape
    m = n // 4
    wt = w.T
    bd = [jnp.kron(jnp.eye(4, dtype=w.dtype),
                   wt[a * h:(a + 1) * h, o * h:(o + 1) * h])
          for a in (0, 1) for o in (0, 1)]
    bz0 = jnp.tile(b[:h], 4).reshape(1, 4 * h)
    bz1 = jnp.tile(b[h:], 4).reshape(1, 4 * h)
    s0p = s0.reshape(m, 4 * h)
    s1p = s1.reshape(m, 4 * h)

    def body(s0r, s1r, b00, b01, b10, b11, z0, z1, o0, o1):
        o0[...] = jnp.maximum(
            jnp.dot(s0r[...], b00[...], preferred_element_type=jnp.float32)
            + jnp.dot(s1r[...], b10[...], preferred_element_type=jnp.float32)
            + z0[...], 0.0)
        o1[...] = jnp.maximum(
            jnp.dot(s0r[...], b01[...], preferred_element_type=jnp.float32)
            + jnp.dot(s1r[...], b11[...], preferred_element_type=jnp.float32)
            + z1[...], 0.0)

    x0p, x1p = pl.pallas_call(
        body,
        out_shape=[jax.ShapeDtypeStruct((m, 4 * h), jnp.float32),
                   jax.ShapeDtypeStruct((m, 4 * h), jnp.float32)],
    )(s0p, s1p, *bd, bz0, bz1)

    x0n, x1n = x0p.reshape(n, h), x1p.reshape(n, h)
    if final:
        return jnp.concatenate([x0n, x1n], axis=1)
    return x0n, x1n


def kernel(z, edge_index, emb, W0, b0, W1, b1, W2, b2):
    n = z.shape[0]
    e = edge_index.shape[1]
    d = emb.shape[1]
    h = d // 2

    n_pad = _round_up(n, NS * CH * 2)
    z_pad = jnp.concatenate(
        [z.astype(jnp.int32), jnp.zeros((n_pad - n,), jnp.int32)])
    x0p, x1p = _sc_embed(z_pad, emb[:, :h], emb[:, h:])
    x0, x1 = x0p[:n], x1p[:n]

    e_ps = -(-e // NS)
    nbl = _round_up(-(-e_ps // (BLK * CH)), 2)
    eps = (nbl + 1) * BLK * CH
    row = jnp.concatenate(
        [edge_index[0].astype(jnp.int32),
         jnp.full((NS * e_ps - e,), n, jnp.int32)]).reshape(NS, e_ps)
    col = jnp.concatenate(
        [edge_index[1].astype(jnp.int32),
         jnp.zeros((NS * e_ps - e,), jnp.int32)]).reshape(NS, e_ps)
    row2d = jnp.concatenate(
        [row, jnp.full((NS, eps - e_ps), n, jnp.int32)],
        axis=1).reshape(-1, CH)
    col2d = jnp.concatenate(
        [col, jnp.zeros((NS, eps - e_ps), jnp.int32)],
        axis=1).reshape(-1, CH)
    sp_rows = _round_up(n + 1, NS)

    for i, (w, b) in enumerate(((W0, b0), (W1, b1), (W2, b2))):
        s0, s1 = _sc_agg(x0, x1, row2d, col2d, n, sp_rows)
        if i < 2:
            x0, x1 = _tc_linear(s0, s1, w, b, final=False)
        else:
            return _tc_linear(s0, s1, w, b, final=True)


# speedup vs baseline: 1.0230x; 1.0230x over previous
import functools

import jax
import jax.numpy as jnp
from jax import lax
from jax.experimental import pallas as pl
from jax.experimental.pallas import tpu as pltpu
from jax.experimental.pallas import tpu_sc as plsc

NC = 2
NS = 16
CH = 128
BLK = 3


def _round_up(x, m):
    return (x + m - 1) // m * m


def _mesh():
    return plsc.VectorSubcoreMesh(
        core_axis_name="c", subcore_axis_name="s",
        num_cores=NC, num_subcores=NS)


def _sc_embed(z_pad, emb0, emb1):
    n_pad, h = z_pad.shape[0], emb0.shape[1]
    chunks = n_pad // (NS * CH)

    @functools.partial(
        pl.kernel,
        out_type=[jax.ShapeDtypeStruct((n_pad, h), jnp.float32),
                  jax.ShapeDtypeStruct((n_pad, h), jnp.float32)],
        mesh=_mesh(),
        compiler_params=pltpu.CompilerParams(use_tc_tiling_on_sc=False),
        scratch_types=[pltpu.VMEM((CH,), jnp.int32),
                       pltpu.VMEM((CH, h), jnp.float32),
                       pltpu.SemaphoreType.DMA],
    )
    def k(z_ref, e0_ref, e1_ref, x0_ref, x1_ref, idx_v, rows_v, sem):
        c = lax.axis_index("c")
        s = lax.axis_index("s")
        base = s * (chunks * CH)

        def body(e_ref, x_ref):
            @pl.loop(0, chunks)
            def _(j):
                off = base + j * CH
                pltpu.sync_copy(z_ref.at[pl.ds(off, CH)], idx_v)
                pltpu.async_copy(e_ref.at[idx_v], rows_v, sem).wait()
                pltpu.sync_copy(rows_v, x_ref.at[pl.ds(off, CH)])

        @pl.when(c == 0)
        def _():
            body(e0_ref, x0_ref)

        @pl.when(c == 1)
        def _():
            body(e1_ref, x1_ref)

    return k(z_pad, emb0, emb1)


def _sc_agg(x0, x1, row2d, col2d, n, sp_rows):
    h = x0.shape[1]
    rps = row2d.shape[0] // NS
    nbl = rps // BLK - 1
    nout = n // NS

    @functools.partial(
        pl.kernel,
        out_type=[jax.ShapeDtypeStruct((n, h), jnp.float32),
                  jax.ShapeDtypeStruct((n, h), jnp.float32)],
        mesh=_mesh(),
        compiler_params=pltpu.CompilerParams(use_tc_tiling_on_sc=False),
        scratch_types=[pltpu.VMEM((BLK, CH), jnp.int32),
                       pltpu.VMEM((BLK, CH), jnp.int32),
                       pltpu.VMEM((BLK, CH), jnp.int32),
                       pltpu.VMEM((BLK, CH), jnp.int32),
                       pltpu.VMEM((BLK, CH, h), jnp.float32),
                       pltpu.VMEM((BLK, CH, h), jnp.float32),
                       pltpu.VMEM_SHARED((sp_rows, h), jnp.float32),
                       pltpu.SemaphoreType.DMA,
                       pltpu.SemaphoreType.DMA,
                       pltpu.SemaphoreType.DMA,
                       pltpu.SemaphoreType.DMA],
    )
    def k(x0_ref, x1_ref, row_ref, col_ref, o0_ref, o1_ref,
          r0b, c0b, r1b, c1b, gb0, gb1, acc,
          sem_g0, sem_g1, sem_s0, sem_s1):
        c = lax.axis_index("c")
        s = lax.axis_index("s")

        def body(x_ref, o_ref):
            pltpu.sync_copy(x_ref.at[pl.ds(s * nout, nout)],
                            acc.at[pl.ds(s * nout, nout)])
            plsc.subcore_barrier()

            base = s * rps

            def load_idx(b, rb, cb):
                pltpu.sync_copy(row_ref.at[pl.ds(base + b * BLK, BLK)], rb)
                pltpu.sync_copy(col_ref.at[pl.ds(base + b * BLK, BLK)], cb)

            def fire_g(cb, gb, sem):
                return [pltpu.async_copy(x_ref.at[cb.at[j]], gb.at[j], sem)
                        for j in range(BLK)]

            def fire_s(rb, gb, sem):
                return [pltpu.async_copy(gb.at[j], acc.at[rb.at[j]], sem,
                                         add=True)
                        for j in range(BLK)]

            load_idx(0, r0b, c0b)

            @pl.loop(0, nbl, step=2)
            def _(i):
                g0 = fire_g(c0b, gb0, sem_g0)
                load_idx(i + 1, r1b, c1b)
                g1 = fire_g(c1b, gb1, sem_g1)
                for dsc in g0:
                    dsc.wait()
                s0 = fire_s(r0b, gb0, sem_s0)
                for dsc in s0:
                    dsc.wait()
                load_idx(i + 2, r0b, c0b)
                for dsc in g1:
                    dsc.wait()
                s1 = fire_s(r1b, gb1, sem_s1)
                for dsc in s1:
                    dsc.wait()

            plsc.subcore_barrier()
            pltpu.sync_copy(acc.at[pl.ds(s * nout, nout)],
                            o_ref.at[pl.ds(s * nout, nout)])

        @pl.when(c == 0)
        def _():
            body(x0_ref, o0_ref)

        @pl.when(c == 1)
        def _():
            body(x1_ref, o1_ref)

    return k(x0, x1, row2d, col2d)


def _tc_linear(s0, s1, w, b, final):
    n, h = s0.shape
    m = n // 4
    wt = w.T
    bd = [jnp.kron(jnp.eye(4, dtype=w.dtype),
                   wt[a * h:(a + 1) * h, o * h:(o + 1) * h])
          for a in (0, 1) for o in (0, 1)]
    bz0 = jnp.tile(b[:h], 4).reshape(1, 4 * h)
    bz1 = jnp.tile(b[h:], 4).reshape(1, 4 * h)
    s0p = s0.reshape(m, 4 * h)
    s1p = s1.reshape(m, 4 * h)

    def body(s0r, s1r, b00, b01, b10, b11, z0, z1, o0, o1):
        o0[...] = jnp.maximum(
            jnp.dot(s0r[...], b00[...], preferred_element_type=jnp.float32)
            + jnp.dot(s1r[...], b10[...], preferred_element_type=jnp.float32)
            + z0[...], 0.0)
        o1[...] = jnp.maximum(
            jnp.dot(s0r[...], b01[...], preferred_element_type=jnp.float32)
            + jnp.dot(s1r[...], b11[...], preferred_element_type=jnp.float32)
            + z1[...], 0.0)

    x0p, x1p = pl.pallas_call(
        body,
        out_shape=[jax.ShapeDtypeStruct((m, 4 * h), jnp.float32),
                   jax.ShapeDtypeStruct((m, 4 * h), jnp.float32)],
    )(s0p, s1p, *bd, bz0, bz1)

    x0n, x1n = x0p.reshape(n, h), x1p.reshape(n, h)
    if final:
        return jnp.concatenate([x0n, x1n], axis=1)
    return x0n, x1n


def kernel(z, edge_index, emb, W0, b0, W1, b1, W2, b2):
    n = z.shape[0]
    e = edge_index.shape[1]
    d = emb.shape[1]
    h = d // 2

    n_pad = _round_up(n, NS * CH)
    z_pad = jnp.concatenate(
        [z.astype(jnp.int32), jnp.zeros((n_pad - n,), jnp.int32)])
    x0p, x1p = _sc_embed(z_pad, emb[:, :h], emb[:, h:])
    x0, x1 = x0p[:n], x1p[:n]

    e_ps = -(-e // NS)
    nbl = _round_up(-(-e_ps // (BLK * CH)), 2)
    eps = (nbl + 1) * BLK * CH
    row = jnp.concatenate(
        [edge_index[0].astype(jnp.int32),
         jnp.full((NS * e_ps - e,), n, jnp.int32)]).reshape(NS, e_ps)
    col = jnp.concatenate(
        [edge_index[1].astype(jnp.int32),
         jnp.zeros((NS * e_ps - e,), jnp.int32)]).reshape(NS, e_ps)
    row2d = jnp.concatenate(
        [row, jnp.full((NS, eps - e_ps), n, jnp.int32)],
        axis=1).reshape(-1, CH)
    col2d = jnp.concatenate(
        [col, jnp.zeros((NS, eps - e_ps), jnp.int32)],
        axis=1).reshape(-1, CH)
    sp_rows = _round_up(n + 1, NS)

    for i, (w, b) in enumerate(((W0, b0), (W1, b1), (W2, b2))):
        s0, s1 = _sc_agg(x0, x1, row2d, col2d, n, sp_rows)
        if i < 2:
            x0, x1 = _tc_linear(s0, s1, w, b, final=False)
        else:
            return _tc_linear(s0, s1, w, b, final=True)
